# Initial kernel scaffold; baseline (speedup 1.0000x reference)
import jax
import jax.numpy as jnp
from jax.experimental import pallas as pl


def kernel(x, edge_index, W1, a1, b1, W2, a2, b2):
    raise NotImplementedError("write your pallas kernel here")


# speedup vs baseline: 5.9635x; 5.9635x over previous
import functools

import jax
import jax.numpy as jnp
from jax import lax
from jax.experimental import pallas as pl
from jax.experimental.pallas import tpu as pltpu
from jax.experimental.pallas import tpu_sc as plsc

N = 4096
E = 65536
EP = E + N
NFEAT = 256
NHID = 32
NHEAD = 8
NCLASS = 10
D1 = NHEAD * NHID
D2 = 16
ALPHA = 0.2
NT = 32
RPT = N // NT
C = 128
NB = 12 + 4 * NT


def _tc1_body(x_ref, w_ref, at_ref, ab_ref, h_ref, s_ref, t_ref, m_ref):
    h = jnp.dot(x_ref[...], w_ref[...], preferred_element_type=jnp.float32)
    h_ref[...] = h
    s = jnp.dot(h, at_ref[...], preferred_element_type=jnp.float32)
    t = jnp.dot(h, ab_ref[...], preferred_element_type=jnp.float32)
    tmax = jnp.max(t, axis=0, keepdims=True)
    sm = s + tmax
    s_ref[...] = s
    t_ref[...] = t
    m_ref[...] = jnp.where(sm > 0, sm, ALPHA * sm)


def _tc2_body(agg_ref, b1_ref, w2_ref, at_ref, ab_ref,
              h2_ref, s_ref, t_ref, m_ref):
    y = agg_ref[...] + b1_ref[...]
    g = jnp.where(y > 0, y, jnp.exp(jnp.minimum(y, 0.0)) - 1.0)
    h2 = jnp.dot(g, w2_ref[...], preferred_element_type=jnp.float32)
    h2_ref[...] = h2
    s = jnp.dot(h2, at_ref[...], preferred_element_type=jnp.float32)
    t = jnp.dot(h2, ab_ref[...], preferred_element_type=jnp.float32)
    tmax = jnp.max(t, axis=0, keepdims=True)
    sm = s + tmax
    s_ref[...] = s
    t_ref[...] = t
    m_ref[...] = jnp.where(sm > 0, sm, ALPHA * sm)


def _tc3_body(agg_ref, b2_ref, out_ref):
    y = agg_ref[...] + b2_ref[...]
    col = lax.broadcasted_iota(jnp.int32, y.shape, 1)
    live = col < NCLASS
    yv = jnp.where(live, y, -1e30)
    mx = jnp.max(yv, axis=1, keepdims=True)
    ex = jnp.where(live, jnp.exp(y - mx), 0.0)
    lse = jnp.log(jnp.sum(ex, axis=1, keepdims=True))
    out_ref[...] = (y - mx - lse)[:, :NCLASS]


_f32 = jnp.float32


def _tc1(x, wcat, atop, abot):
    return pl.pallas_call(
        _tc1_body,
        out_shape=(
            jax.ShapeDtypeStruct((N, D1), _f32),
            jax.ShapeDtypeStruct((N, NHEAD), _f32),
            jax.ShapeDtypeStruct((N, NHEAD), _f32),
            jax.ShapeDtypeStruct((N, NHEAD), _f32),
        ),
    )(x, wcat, atop, abot)


def _tc2(agg, b1cat, w2p, a2t, a2b):
    return pl.pallas_call(
        _tc2_body,
        out_shape=(
            jax.ShapeDtypeStruct((N, D2), _f32),
            jax.ShapeDtypeStruct((N, NHEAD), _f32),
            jax.ShapeDtypeStruct((N, NHEAD), _f32),
            jax.ShapeDtypeStruct((N, NHEAD), _f32),
        ),
    )(agg, b1cat, w2p, a2t, a2b)


def _tc3(agg2, b2p):
    return pl.pallas_call(
        _tc3_body,
        out_shape=jax.ShapeDtypeStruct((N, NCLASS), _f32),
    )(agg2, b2p)


def _make_sc(K, D):
    HB = D // K
    mesh = plsc.VectorSubcoreMesh(core_axis_name="c", subcore_axis_name="s")

    @functools.partial(
        pl.kernel,
        mesh=mesh,
        compiler_params=pltpu.CompilerParams(
            needs_layout_passes=False, use_tc_tiling_on_sc=False),
        out_type=jax.ShapeDtypeStruct((N, D), _f32),
        scratch_types=[
            pltpu.VMEM((N * NHEAD,), _f32),
            pltpu.VMEM((RPT * NHEAD,), _f32),
            pltpu.VMEM((RPT * NHEAD,), _f32),
            pltpu.VMEM((RPT * NHEAD,), _f32),
            pltpu.VMEM((RPT, D), _f32),
            pltpu.VMEM((C,), jnp.int32),
            pltpu.VMEM((C,), jnp.int32),
            pltpu.VMEM((C, D), _f32),
            pltpu.VMEM((64,), jnp.int32),
            pltpu.SemaphoreType.DMA,
        ],
    )
    def sc_kernel(ekey_hbm, tgt_hbm, s_hbm, t_hbm, m_hbm, h_hbm, starts_hbm,
                  out_hbm, t_v, s_v, m_v, z_v, acc_v, kbuf, ibuf, rowbuf,
                  st_v, sem):
        wid = lax.axis_index("s") * 2 + lax.axis_index("c")
        base = wid * RPT
        iota = lax.iota(jnp.int32, 16)

        pltpu.sync_copy(t_hbm, t_v)
        pltpu.sync_copy(s_hbm.at[pl.ds(base * NHEAD, RPT * NHEAD)], s_v)
        pltpu.sync_copy(m_hbm.at[pl.ds(base * NHEAD, RPT * NHEAD)], m_v)
        pltpu.sync_copy(starts_hbm, st_v)
        start = st_v[pl.ds(wid, 16)][0]
        end = st_v[pl.ds(wid + 1, 16)][0]

        zeros = jnp.zeros((16,), _f32)

        def _z1(i, _):
            z_v[pl.ds(i * 16, 16)] = zeros
            return 0
        lax.fori_loop(0, RPT * NHEAD // 16, _z1, 0)

        def _z2(i, _):
            r = i // (D // 16)
            c = (i % (D // 16)) * 16
            acc_v[r, pl.ds(c, 16)] = zeros
            return 0
        lax.fori_loop(0, RPT * D // 16, _z2, 0)

        j0 = start // C
        j1 = (end + C - 1) // C

        def _decode(j, v):
            kvec = kbuf[pl.ds(v * 16, 16)]
            eidx = (j * C + v * 16) + iota
            dup = (kvec >> 24) > 0
            valid = (eidx >= start) & (eidx < end) & jnp.logical_not(dup)
            srcl = jnp.clip((kvec >> 12) & (N - 1), base, base + RPT - 1) - base
            tgt = kvec & (N - 1)
            return valid, srcl, tgt

        def _edge_p(k, srcl, tgt, valid):
            kk = jnp.full((16,), k, jnp.int32)
            tval = plsc.load_gather(t_v, [tgt * NHEAD + kk])
            sval = plsc.load_gather(s_v, [srcl * NHEAD + kk])
            mval = plsc.load_gather(m_v, [srcl * NHEAD + kk])
            xx = sval + tval
            lg = jnp.where(xx > 0, xx, ALPHA * xx)
            p = jnp.exp(lg - mval)
            return kk, jnp.where(valid, p, 0.0)

        def _chunk1(j, _):
            pltpu.sync_copy(ekey_hbm.at[pl.ds(j * C, C)], kbuf)

            def _vreg(v, _):
                valid, srcl, tgt = _decode(j, v)
                for k in range(K):
                    kk, p = _edge_p(k, srcl, tgt, valid)
                    plsc.addupdate_scatter(
                        z_v, [srcl * NHEAD + kk], p, mask=valid)
                return 0

            lax.fori_loop(0, C // 16, _vreg, 0)
            return 0

        lax.fori_loop(j0, j1, _chunk1, 0)

        def _chunk2(j, _):
            pltpu.sync_copy(ekey_hbm.at[pl.ds(j * C, C)], kbuf)
            pltpu.sync_copy(tgt_hbm.at[pl.ds(j * C, C)], ibuf)
            pltpu.async_copy(h_hbm.at[ibuf], rowbuf, sem).wait()

            def _vreg(v, _):
                valid, srcl, tgt = _decode(j, v)
                rows = v * 16 + iota
                for k in range(K):
                    kk, p = _edge_p(k, srcl, tgt, valid)
                    zval = plsc.load_gather(z_v, [srcl * NHEAD + kk])
                    w = p / zval
                    for c in range(HB):
                        col = jnp.full((16,), k * HB + c, jnp.int32)
                        hval = plsc.load_gather(rowbuf, [rows, col])
                        plsc.addupdate_scatter(
                            acc_v, [srcl, col], w * hval, mask=valid)
                return 0

            lax.fori_loop(0, C // 16, _vreg, 0)
            return 0

        lax.fori_loop(j0, j1, _chunk2, 0)

        pltpu.sync_copy(acc_v, out_hbm.at[pl.ds(base, RPT)])

    return sc_kernel


_sc_l1 = _make_sc(NHEAD, D1)
_sc_l2 = _make_sc(1, D2)


def kernel(x, edge_index, W1, a1, b1, W2, a2, b2):
    i32 = jnp.int32
    f32 = jnp.float32
    x = x.astype(f32)

    loop = jnp.arange(N, dtype=i32)
    srcs = jnp.concatenate([edge_index[0].astype(i32), loop])
    tgts = jnp.concatenate([edge_index[1].astype(i32), loop])
    key = jnp.sort(srcs * N + tgts)
    dupbit = jnp.concatenate(
        [jnp.zeros((1,), i32), (key[1:] == key[:-1]).astype(i32)])
    ekey = key | (dupbit << 24)
    tgtarr = key & (N - 1)
    bounds = (jnp.arange(NT + 1, dtype=i32) * RPT) * N
    starts = jnp.searchsorted(key, bounds).astype(i32)
    starts = jnp.concatenate(
        [starts, jnp.full((64 - NT - 1,), EP, i32)])

    eye = jnp.eye(NHEAD, dtype=f32)
    w1cat = W1.astype(f32).transpose(1, 0, 2).reshape(NFEAT, D1)
    atop = (a1[:, :NHID, 0].astype(f32)[:, :, None]
            * eye[:, None, :]).reshape(D1, NHEAD)
    abot = (a1[:, NHID:, 0].astype(f32)[:, :, None]
            * eye[:, None, :]).reshape(D1, NHEAD)
    b1cat = b1.astype(f32).reshape(1, D1)
    w2p = jnp.concatenate(
        [W2.astype(f32), jnp.zeros((D1, D2 - NCLASS), f32)], axis=1)
    a2t = jnp.zeros((D2, NHEAD), f32).at[:NCLASS, :].set(
        a2[:NCLASS, 0].astype(f32)[:, None])
    a2b = jnp.zeros((D2, NHEAD), f32).at[:NCLASS, :].set(
        a2[NCLASS:, 0].astype(f32)[:, None])
    b2p = jnp.concatenate(
        [b2.astype(f32), jnp.zeros((D2 - NCLASS,), f32)]).reshape(1, D2)

    h1, s1, t1, m1 = _tc1(x, w1cat, atop, abot)
    agg1 = _sc_l1(ekey, tgtarr, s1.reshape(-1), t1.reshape(-1),
                  m1.reshape(-1), h1, starts)
    h2, s2, t2, m2 = _tc2(agg1, b1cat, w2p, a2t, a2b)
    agg2 = _sc_l2(ekey, tgtarr, s2.reshape(-1), t2.reshape(-1),
                  m2.reshape(-1), h2, starts)
    return _tc3(agg2, b2p)


# speedup vs baseline: 15.4886x; 2.5972x over previous
import functools

import jax
import jax.numpy as jnp
from jax import lax
from jax.experimental import pallas as pl
from jax.experimental.pallas import tpu as pltpu
from jax.experimental.pallas import tpu_sc as plsc

N = 4096
E = 65536
EP = E + N
NFEAT = 256
NHID = 32
NHEAD = 8
NCLASS = 10
D1 = NHEAD * NHID
D2 = 16
ALPHA = 0.2
NT = 32
RPT = N // NT
C = 128
NB = 12 + 4 * NT


def _tc1_body(x_ref, w_ref, at_ref, ab_ref, h_ref, s_ref, t_ref, m_ref):
    h = jnp.dot(x_ref[...], w_ref[...], preferred_element_type=jnp.float32)
    h_ref[...] = h
    s = jnp.dot(h, at_ref[...], preferred_element_type=jnp.float32)
    t = jnp.dot(h, ab_ref[...], preferred_element_type=jnp.float32)
    tmax = jnp.max(t, axis=0, keepdims=True)
    sm = s + tmax
    s_ref[...] = s
    t_ref[...] = t
    m_ref[...] = jnp.where(sm > 0, sm, ALPHA * sm)


def _tc2_body(agg_ref, b1_ref, w2_ref, at_ref, ab_ref,
              h2_ref, s_ref, t_ref, m_ref):
    y = agg_ref[...] + b1_ref[...]
    g = jnp.where(y > 0, y, jnp.exp(jnp.minimum(y, 0.0)) - 1.0)
    h2 = jnp.dot(g, w2_ref[...], preferred_element_type=jnp.float32)
    h2_ref[...] = h2
    s = jnp.dot(h2, at_ref[...], preferred_element_type=jnp.float32)
    t = jnp.dot(h2, ab_ref[...], preferred_element_type=jnp.float32)
    tmax = jnp.max(t, axis=0, keepdims=True)
    sm = s + tmax
    s_ref[...] = s
    t_ref[...] = t
    m_ref[...] = jnp.where(sm > 0, sm, ALPHA * sm)


def _tc3_body(agg_ref, b2_ref, out_ref):
    y = agg_ref[...] + b2_ref[...]
    col = lax.broadcasted_iota(jnp.int32, y.shape, 1)
    live = col < NCLASS
    yv = jnp.where(live, y, -1e30)
    mx = jnp.max(yv, axis=1, keepdims=True)
    ex = jnp.where(live, jnp.exp(y - mx), 0.0)
    lse = jnp.log(jnp.sum(ex, axis=1, keepdims=True))
    out_ref[...] = (y - mx - lse)[:, :NCLASS]


_f32 = jnp.float32


def _tc1(x, wcat, atop, abot):
    return pl.pallas_call(
        _tc1_body,
        out_shape=(
            jax.ShapeDtypeStruct((N, D1), _f32),
            jax.ShapeDtypeStruct((N, NHEAD), _f32),
            jax.ShapeDtypeStruct((N, NHEAD), _f32),
            jax.ShapeDtypeStruct((N, NHEAD), _f32),
        ),
    )(x, wcat, atop, abot)


def _tc2(agg, b1cat, w2p, a2t, a2b):
    return pl.pallas_call(
        _tc2_body,
        out_shape=(
            jax.ShapeDtypeStruct((N, D2), _f32),
            jax.ShapeDtypeStruct((N, NHEAD), _f32),
            jax.ShapeDtypeStruct((N, NHEAD), _f32),
            jax.ShapeDtypeStruct((N, NHEAD), _f32),
        ),
    )(agg, b1cat, w2p, a2t, a2b)


def _tc3(agg2, b2p):
    return pl.pallas_call(
        _tc3_body,
        out_shape=jax.ShapeDtypeStruct((N, NCLASS), _f32),
    )(agg2, b2p)


def _make_sc(K, D):
    HB = D // K
    mesh = plsc.VectorSubcoreMesh(core_axis_name="c", subcore_axis_name="s")

    @functools.partial(
        pl.kernel,
        mesh=mesh,
        compiler_params=pltpu.CompilerParams(
            needs_layout_passes=False, use_tc_tiling_on_sc=False),
        out_type=jax.ShapeDtypeStruct((N * D,), _f32),
        scratch_types=[
            pltpu.VMEM((N * NHEAD,), _f32),
            pltpu.VMEM((RPT * NHEAD,), _f32),
            pltpu.VMEM((RPT * NHEAD,), _f32),
            pltpu.VMEM((RPT * NHEAD,), _f32),
            pltpu.VMEM((RPT * D,), _f32),
            pltpu.VMEM((C,), jnp.int32),
            pltpu.VMEM((C,), jnp.int32),
            pltpu.VMEM((C, D), _f32),
            pltpu.VMEM((64,), jnp.int32),
            pltpu.SemaphoreType.DMA,
        ],
    )
    def sc_kernel(ekey_hbm, tgt_hbm, s_hbm, t_hbm, m_hbm, h_hbm, starts_hbm,
                  out_hbm, t_v, s_v, m_v, z_v, acc_v, kbuf, ibuf, rowbuf,
                  st_v, sem):
        wid = lax.axis_index("s") * 2 + lax.axis_index("c")
        base = wid * RPT
        iota = lax.iota(jnp.int32, 16)

        pltpu.sync_copy(t_hbm, t_v)
        pltpu.sync_copy(s_hbm.at[pl.ds(base * NHEAD, RPT * NHEAD)], s_v)
        pltpu.sync_copy(m_hbm.at[pl.ds(base * NHEAD, RPT * NHEAD)], m_v)
        pltpu.sync_copy(starts_hbm, st_v)
        start = st_v[pl.ds(wid, 16)][0]
        end = st_v[pl.ds(wid + 1, 16)][0]

        zeros = jnp.zeros((16,), _f32)

        def _z1(i, _):
            z_v[pl.ds(i * 16, 16)] = zeros
            return 0
        lax.fori_loop(0, RPT * NHEAD // 16, _z1, 0)

        def _z2(i, _):
            acc_v[pl.ds(i * 16, 16)] = zeros
            return 0
        lax.fori_loop(0, RPT * D // 16, _z2, 0)

        j0 = start // C
        j1 = (end + C - 1) // C

        def _decode(j, v):
            kvec = kbuf[pl.ds(v * 16, 16)]
            eidx = (j * C + v * 16) + iota
            dup = (kvec >> 24) > 0
            valid = (eidx >= start) & (eidx < end) & jnp.logical_not(dup)
            srcl = jnp.clip((kvec >> 12) & (N - 1), base, base + RPT - 1) - base
            tgt = kvec & (N - 1)
            return valid, srcl, tgt

        def _edge_p(k, srcl, tgt, valid):
            kk = jnp.full((16,), k, jnp.int32)
            tval = plsc.load_gather(t_v, [tgt * NHEAD + kk])
            sval = plsc.load_gather(s_v, [srcl * NHEAD + kk])
            mval = plsc.load_gather(m_v, [srcl * NHEAD + kk])
            xx = sval + tval
            lg = jnp.where(xx > 0, xx, ALPHA * xx)
            p = jnp.exp(lg - mval)
            return kk, jnp.where(valid, p, 0.0)

        def _chunk1(j, _):
            pltpu.sync_copy(ekey_hbm.at[pl.ds(j * C, C)], kbuf)

            def _vreg(v, _):
                valid, srcl, tgt = _decode(j, v)
                for k in range(K):
                    kk, p = _edge_p(k, srcl, tgt, valid)
                    plsc.addupdate_scatter(
                        z_v, [srcl * NHEAD + kk], p, mask=valid)
                return 0

            lax.fori_loop(0, C // 16, _vreg, 0)
            return 0

        lax.fori_loop(j0, j1, _chunk1, 0)

        def _chunk2(j, _):
            pltpu.sync_copy(ekey_hbm.at[pl.ds(j * C, C)], kbuf)
            pltpu.sync_copy(tgt_hbm.at[pl.ds(j * C, C)], ibuf)
            pltpu.async_copy(h_hbm.at[ibuf], rowbuf, sem).wait()

            def _vreg(v, _):
                valid, srcl, tgt = _decode(j, v)
                ws = []
                for k in range(K):
                    kk, p = _edge_p(k, srcl, tgt, valid)
                    zval = plsc.load_gather(z_v, [srcl * NHEAD + kk])
                    ws.append(p / zval)
                off_vec = srcl * D
                for i in range(16):
                    off = off_vec[i]
                    row = v * 16 + i
                    for k in range(K):
                        wsc = ws[k][i]
                        for c2 in range(HB // 16):
                            cc = k * HB + c2 * 16
                            hv = rowbuf[row, pl.ds(cc, 16)]
                            plsc.addupdate(
                                acc_v.at[pl.ds(off + cc, 16)], hv * wsc)
                return 0

            lax.fori_loop(0, C // 16, _vreg, 0)
            return 0

        lax.fori_loop(j0, j1, _chunk2, 0)

        pltpu.sync_copy(acc_v, out_hbm.at[pl.ds(base * D, RPT * D)])

    return sc_kernel


_sc_l1 = _make_sc(NHEAD, D1)
_sc_l2 = _make_sc(1, D2)


def kernel(x, edge_index, W1, a1, b1, W2, a2, b2):
    i32 = jnp.int32
    f32 = jnp.float32
    x = x.astype(f32)

    loop = jnp.arange(N, dtype=i32)
    srcs = jnp.concatenate([edge_index[0].astype(i32), loop])
    tgts = jnp.concatenate([edge_index[1].astype(i32), loop])
    key = jnp.sort(srcs * N + tgts)
    dupbit = jnp.concatenate(
        [jnp.zeros((1,), i32), (key[1:] == key[:-1]).astype(i32)])
    ekey = key | (dupbit << 24)
    tgtarr = key & (N - 1)
    bounds = (jnp.arange(NT + 1, dtype=i32) * RPT) * N
    starts = jnp.searchsorted(key, bounds).astype(i32)
    starts = jnp.concatenate(
        [starts, jnp.full((64 - NT - 1,), EP, i32)])

    eye = jnp.eye(NHEAD, dtype=f32)
    w1cat = W1.astype(f32).transpose(1, 0, 2).reshape(NFEAT, D1)
    atop = (a1[:, :NHID, 0].astype(f32)[:, :, None]
            * eye[:, None, :]).reshape(D1, NHEAD)
    abot = (a1[:, NHID:, 0].astype(f32)[:, :, None]
            * eye[:, None, :]).reshape(D1, NHEAD)
    b1cat = b1.astype(f32).reshape(1, D1)
    w2p = jnp.concatenate(
        [W2.astype(f32), jnp.zeros((D1, D2 - NCLASS), f32)], axis=1)
    a2t = jnp.zeros((D2, NHEAD), f32).at[:NCLASS, :].set(
        a2[:NCLASS, 0].astype(f32)[:, None])
    a2b = jnp.zeros((D2, NHEAD), f32).at[:NCLASS, :].set(
        a2[NCLASS:, 0].astype(f32)[:, None])
    b2p = jnp.concatenate(
        [b2.astype(f32), jnp.zeros((D2 - NCLASS,), f32)]).reshape(1, D2)

    h1, s1, t1, m1 = _tc1(x, w1cat, atop, abot)
    agg1 = _sc_l1(ekey, tgtarr, s1.reshape(-1), t1.reshape(-1),
                  m1.reshape(-1), h1, starts).reshape(N, D1)
    h2, s2, t2, m2 = _tc2(agg1, b1cat, w2p, a2t, a2b)
    agg2 = _sc_l2(ekey, tgtarr, s2.reshape(-1), t2.reshape(-1),
                  m2.reshape(-1), h2, starts).reshape(N, D2)
    return _tc3(agg2, b2p)


# speedup vs baseline: 16.7263x; 1.0799x over previous
import functools

import jax
import jax.numpy as jnp
from jax import lax
from jax.experimental import pallas as pl
from jax.experimental.pallas import tpu as pltpu
from jax.experimental.pallas import tpu_sc as plsc

N = 4096
E = 65536
EP = E + N
NFEAT = 256
NHID = 32
NHEAD = 8
NCLASS = 10
D1 = NHEAD * NHID
D2 = 16
ALPHA = 0.2
NT = 32
RPT = N // NT
C = 96
EPAD = ((EP + C - 1) // C) * C
JLAST = EPAD // C - 1
NB = 12 + 4 * NT


def _tc1_body(x_ref, w_ref, at_ref, ab_ref, h_ref, s_ref, t_ref, m_ref):
    h = jnp.dot(x_ref[...], w_ref[...], preferred_element_type=jnp.float32)
    h_ref[...] = h
    s = jnp.dot(h, at_ref[...], preferred_element_type=jnp.float32)
    t = jnp.dot(h, ab_ref[...], preferred_element_type=jnp.float32)
    tmax = jnp.max(t, axis=0, keepdims=True)
    sm = s + tmax
    s_ref[...] = s
    t_ref[...] = t
    m_ref[...] = jnp.where(sm > 0, sm, ALPHA * sm)


def _tc2_body(agg_ref, b1_ref, w2_ref, at_ref, ab_ref,
              h2_ref, s_ref, t_ref, m_ref):
    y = agg_ref[...] + b1_ref[...]
    g = jnp.where(y > 0, y, jnp.exp(jnp.minimum(y, 0.0)) - 1.0)
    h2 = jnp.dot(g, w2_ref[...], preferred_element_type=jnp.float32)
    h2_ref[...] = h2
    s = jnp.dot(h2, at_ref[...], preferred_element_type=jnp.float32)
    t = jnp.dot(h2, ab_ref[...], preferred_element_type=jnp.float32)
    tmax = jnp.max(t, axis=0, keepdims=True)
    sm = s + tmax
    s_ref[...] = s
    t_ref[...] = t
    m_ref[...] = jnp.where(sm > 0, sm, ALPHA * sm)


def _tc3_body(agg_ref, b2_ref, out_ref):
    y = agg_ref[...] + b2_ref[...]
    col = lax.broadcasted_iota(jnp.int32, y.shape, 1)
    live = col < NCLASS
    yv = jnp.where(live, y, -1e30)
    mx = jnp.max(yv, axis=1, keepdims=True)
    ex = jnp.where(live, jnp.exp(y - mx), 0.0)
    lse = jnp.log(jnp.sum(ex, axis=1, keepdims=True))
    out_ref[...] = (y - mx - lse)[:, :NCLASS]


_f32 = jnp.float32


def _tc1(x, wcat, atop, abot):
    return pl.pallas_call(
        _tc1_body,
        out_shape=(
            jax.ShapeDtypeStruct((N, D1), _f32),
            jax.ShapeDtypeStruct((N, NHEAD), _f32),
            jax.ShapeDtypeStruct((N, NHEAD), _f32),
            jax.ShapeDtypeStruct((N, NHEAD), _f32),
        ),
    )(x, wcat, atop, abot)


def _tc2(agg, b1cat, w2p, a2t, a2b):
    return pl.pallas_call(
        _tc2_body,
        out_shape=(
            jax.ShapeDtypeStruct((N, D2), _f32),
            jax.ShapeDtypeStruct((N, NHEAD), _f32),
            jax.ShapeDtypeStruct((N, NHEAD), _f32),
            jax.ShapeDtypeStruct((N, NHEAD), _f32),
        ),
    )(agg, b1cat, w2p, a2t, a2b)


def _tc3(agg2, b2p):
    return pl.pallas_call(
        _tc3_body,
        out_shape=jax.ShapeDtypeStruct((N, NCLASS), _f32),
    )(agg2, b2p)


def _make_sc(K, D):
    HB = D // K
    mesh = plsc.VectorSubcoreMesh(core_axis_name="c", subcore_axis_name="s")

    @functools.partial(
        pl.kernel,
        mesh=mesh,
        compiler_params=pltpu.CompilerParams(
            needs_layout_passes=False, use_tc_tiling_on_sc=False),
        out_type=jax.ShapeDtypeStruct((N * D,), _f32),
        scratch_types=[
            pltpu.VMEM((N * NHEAD,), _f32),
            pltpu.VMEM((RPT * NHEAD,), _f32),
            pltpu.VMEM((RPT * NHEAD,), _f32),
            pltpu.VMEM((RPT * NHEAD,), _f32),
            pltpu.VMEM((RPT * D,), _f32),
            pltpu.VMEM((C,), jnp.int32),
            pltpu.VMEM((C,), jnp.int32),
            pltpu.VMEM((C,), jnp.int32),
            pltpu.VMEM((C,), jnp.int32),
            pltpu.VMEM((C, D), _f32),
            pltpu.VMEM((C, D), _f32),
            pltpu.VMEM((64,), jnp.int32),
            pltpu.SemaphoreType.DMA,
            pltpu.SemaphoreType.DMA,
        ],
    )
    def sc_kernel(ekey_hbm, tgt_hbm, s_hbm, t_hbm, m_hbm, h_hbm, starts_hbm,
                  out_hbm, t_v, s_v, m_v, z_v, acc_v, kbuf0, kbuf1, ibuf0,
                  ibuf1, rowbuf0, rowbuf1, st_v, sem0, sem1):
        wid = lax.axis_index("s") * 2 + lax.axis_index("c")
        base = wid * RPT
        iota = lax.iota(jnp.int32, 16)

        pltpu.sync_copy(t_hbm, t_v)
        pltpu.sync_copy(s_hbm.at[pl.ds(base * NHEAD, RPT * NHEAD)], s_v)
        pltpu.sync_copy(m_hbm.at[pl.ds(base * NHEAD, RPT * NHEAD)], m_v)
        pltpu.sync_copy(starts_hbm, st_v)
        start = st_v[pl.ds(wid, 16)][0]
        end = st_v[pl.ds(wid + 1, 16)][0]

        zeros = jnp.zeros((16,), _f32)

        def _z1(i, _):
            z_v[pl.ds(i * 16, 16)] = zeros
            return 0
        lax.fori_loop(0, RPT * NHEAD // 16, _z1, 0)

        def _z2(i, _):
            acc_v[pl.ds(i * 16, 16)] = zeros
            return 0
        lax.fori_loop(0, RPT * D // 16, _z2, 0)

        j0 = start // C
        j1 = (end + C - 1) // C
        npairs = (j1 - j0 + 1) // 2

        def _decode(j, kb, v):
            kvec = kb[pl.ds(v * 16, 16)]
            eidx = (j * C + v * 16) + iota
            dup = (kvec >> 24) > 0
            valid = (eidx >= start) & (eidx < end) & jnp.logical_not(dup)
            srcl = jnp.clip((kvec >> 12) & (N - 1), base, base + RPT - 1) - base
            tgt = kvec & (N - 1)
            return valid, srcl, tgt

        def _edge_p(k, srcl, tgt, valid):
            kk = jnp.full((16,), k, jnp.int32)
            tval = plsc.load_gather(t_v, [tgt * NHEAD + kk])
            sval = plsc.load_gather(s_v, [srcl * NHEAD + kk])
            mval = plsc.load_gather(m_v, [srcl * NHEAD + kk])
            xx = sval + tval
            lg = jnp.where(xx > 0, xx, ALPHA * xx)
            p = jnp.exp(lg - mval)
            return kk, jnp.where(valid, p, 0.0)

        def _p1_load(jc, kb, sm):
            pltpu.async_copy(ekey_hbm.at[pl.ds(jc * C, C)], kb, sm)

        def _p1_wait(kb, sm):
            pltpu.make_async_copy(ekey_hbm.at[pl.ds(0, C)], kb, sm).wait()

        def _p1_proc(j, kb):
            def _vreg(v, _):
                valid, srcl, tgt = _decode(j, kb, v)
                for k in range(K):
                    kk, p = _edge_p(k, srcl, tgt, valid)
                    plsc.addupdate_scatter(
                        z_v, [srcl * NHEAD + kk], p, mask=valid)
                return 0
            lax.fori_loop(0, C // 16, _vreg, 0)

        _p1_load(j0, kbuf0, sem0)

        def _pair1(s_, _):
            j = j0 + 2 * s_
            _p1_load(jnp.minimum(j + 1, JLAST), kbuf1, sem1)
            _p1_wait(kbuf0, sem0)
            _p1_proc(j, kbuf0)
            _p1_load(jnp.minimum(j + 2, JLAST), kbuf0, sem0)
            _p1_wait(kbuf1, sem1)
            _p1_proc(j + 1, kbuf1)
            return 0

        lax.fori_loop(0, npairs, _pair1, 0)
        _p1_wait(kbuf0, sem0)

        def _p2_load(jc, kb, ib, rb, sm):
            pltpu.sync_copy(ekey_hbm.at[pl.ds(jc * C, C)], kb)
            pltpu.sync_copy(tgt_hbm.at[pl.ds(jc * C, C)], ib)
            pltpu.async_copy(h_hbm.at[ib], rb, sm)

        def _p2_wait(ib, rb, sm):
            pltpu.make_async_copy(h_hbm.at[ib], rb, sm).wait()

        def _p2_proc(j, kb, rb):
            def _vreg(v, _):
                valid, srcl, tgt = _decode(j, kb, v)
                ws = []
                for k in range(K):
                    kk, p = _edge_p(k, srcl, tgt, valid)
                    zval = plsc.load_gather(z_v, [srcl * NHEAD + kk])
                    ws.append(p / zval)
                off_vec = srcl * D
                for i in range(16):
                    off = off_vec[i]
                    row = v * 16 + i
                    for k in range(K):
                        wsc = ws[k][i]
                        for c2 in range(HB // 16):
                            cc = k * HB + c2 * 16
                            hv = rb[row, pl.ds(cc, 16)]
                            plsc.addupdate(
                                acc_v.at[pl.ds(off + cc, 16)], hv * wsc)
                return 0
            lax.fori_loop(0, C // 16, _vreg, 0)

        _p2_load(j0, kbuf0, ibuf0, rowbuf0, sem0)

        def _pair2(s_, _):
            j = j0 + 2 * s_
            _p2_load(jnp.minimum(j + 1, JLAST), kbuf1, ibuf1, rowbuf1, sem1)
            _p2_wait(ibuf0, rowbuf0, sem0)
            _p2_proc(j, kbuf0, rowbuf0)
            _p2_load(jnp.minimum(j + 2, JLAST), kbuf0, ibuf0, rowbuf0, sem0)
            _p2_wait(ibuf1, rowbuf1, sem1)
            _p2_proc(j + 1, kbuf1, rowbuf1)
            return 0

        lax.fori_loop(0, npairs, _pair2, 0)
        _p2_wait(ibuf0, rowbuf0, sem0)

        pltpu.sync_copy(acc_v, out_hbm.at[pl.ds(base * D, RPT * D)])

    return sc_kernel


_sc_l1 = _make_sc(NHEAD, D1)
_sc_l2 = _make_sc(1, D2)


def kernel(x, edge_index, W1, a1, b1, W2, a2, b2):
    i32 = jnp.int32
    f32 = jnp.float32
    x = x.astype(f32)

    loop = jnp.arange(N, dtype=i32)
    srcs = jnp.concatenate([edge_index[0].astype(i32), loop])
    tgts = jnp.concatenate([edge_index[1].astype(i32), loop])
    key = jnp.sort(srcs * N + tgts)
    dupbit = jnp.concatenate(
        [jnp.zeros((1,), i32), (key[1:] == key[:-1]).astype(i32)])
    ekey = jnp.concatenate(
        [key | (dupbit << 24), jnp.full((EPAD - EP,), 1 << 24, i32)])
    tgtarr = jnp.concatenate(
        [key & (N - 1), jnp.zeros((EPAD - EP,), i32)])
    bounds = (jnp.arange(NT + 1, dtype=i32) * RPT) * N
    starts = jnp.searchsorted(key, bounds).astype(i32)
    starts = jnp.concatenate(
        [starts, jnp.full((64 - NT - 1,), EP, i32)])

    eye = jnp.eye(NHEAD, dtype=f32)
    w1cat = W1.astype(f32).transpose(1, 0, 2).reshape(NFEAT, D1)
    atop = (a1[:, :NHID, 0].astype(f32)[:, :, None]
            * eye[:, None, :]).reshape(D1, NHEAD)
    abot = (a1[:, NHID:, 0].astype(f32)[:, :, None]
            * eye[:, None, :]).reshape(D1, NHEAD)
    b1cat = b1.astype(f32).reshape(1, D1)
    w2p = jnp.concatenate(
        [W2.astype(f32), jnp.zeros((D1, D2 - NCLASS), f32)], axis=1)
    a2t = jnp.zeros((D2, NHEAD), f32).at[:NCLASS, :].set(
        a2[:NCLASS, 0].astype(f32)[:, None])
    a2b = jnp.zeros((D2, NHEAD), f32).at[:NCLASS, :].set(
        a2[NCLASS:, 0].astype(f32)[:, None])
    b2p = jnp.concatenate(
        [b2.astype(f32), jnp.zeros((D2 - NCLASS,), f32)]).reshape(1, D2)

    h1, s1, t1, m1 = _tc1(x, w1cat, atop, abot)
    agg1 = _sc_l1(ekey, tgtarr, s1.reshape(-1), t1.reshape(-1),
                  m1.reshape(-1), h1, starts).reshape(N, D1)
    h2, s2, t2, m2 = _tc2(agg1, b1cat, w2p, a2t, a2b)
    agg2 = _sc_l2(ekey, tgtarr, s2.reshape(-1), t2.reshape(-1),
                  m2.reshape(-1), h2, starts).reshape(N, D2)
    return _tc3(agg2, b2p)


# speedup vs baseline: 24.1428x; 1.4434x over previous
import functools

import jax
import jax.numpy as jnp
from jax import lax
from jax.experimental import pallas as pl
from jax.experimental.pallas import tpu as pltpu
from jax.experimental.pallas import tpu_sc as plsc

N = 4096
E = 65536
EP = E + N
NFEAT = 256
NHID = 32
NHEAD = 8
NCLASS = 10
D1 = NHEAD * NHID
D2 = 16
ALPHA = 0.2
NT = 32
RPT = N // NT
C = 96
EPAD = ((EP + C - 1) // C) * C
JLAST = EPAD // C - 1
NB = 12 + 4 * NT


def _tc1_body(x_ref, w_ref, at_ref, ab_ref, h_ref, s_ref, t_ref, m_ref):
    h = jnp.dot(x_ref[...], w_ref[...], preferred_element_type=jnp.float32)
    h_ref[...] = h
    s = jnp.dot(h, at_ref[...], preferred_element_type=jnp.float32)
    t = jnp.dot(h, ab_ref[...], preferred_element_type=jnp.float32)
    tmax = jnp.max(t, axis=0, keepdims=True)
    sm = s + tmax
    s_ref[...] = s
    t_ref[...] = t
    m_ref[...] = jnp.where(sm > 0, sm, ALPHA * sm)


def _tc2_body(agg_ref, b1_ref, w2_ref, at_ref, ab_ref,
              h2_ref, s_ref, t_ref, m_ref):
    y = agg_ref[...] + b1_ref[...]
    g = jnp.where(y > 0, y, jnp.exp(jnp.minimum(y, 0.0)) - 1.0)
    h2 = jnp.dot(g, w2_ref[...], preferred_element_type=jnp.float32)
    h2_ref[...] = h2
    s = jnp.dot(h2, at_ref[...], preferred_element_type=jnp.float32)
    t = jnp.dot(h2, ab_ref[...], preferred_element_type=jnp.float32)
    tmax = jnp.max(t, axis=0, keepdims=True)
    sm = s + tmax
    s_ref[...] = s
    t_ref[...] = t
    m_ref[...] = jnp.where(sm > 0, sm, ALPHA * sm)


def _tc3_body(agg_ref, b2_ref, out_ref):
    y = agg_ref[...] + b2_ref[...]
    col = lax.broadcasted_iota(jnp.int32, y.shape, 1)
    live = col < NCLASS
    yv = jnp.where(live, y, -1e30)
    mx = jnp.max(yv, axis=1, keepdims=True)
    ex = jnp.where(live, jnp.exp(y - mx), 0.0)
    lse = jnp.log(jnp.sum(ex, axis=1, keepdims=True))
    out_ref[...] = (y - mx - lse)[:, :NCLASS]


_f32 = jnp.float32


def _tc1(x, wcat, atop, abot):
    return pl.pallas_call(
        _tc1_body,
        out_shape=(
            jax.ShapeDtypeStruct((N, D1), _f32),
            jax.ShapeDtypeStruct((N, NHEAD), _f32),
            jax.ShapeDtypeStruct((N, NHEAD), _f32),
            jax.ShapeDtypeStruct((N, NHEAD), _f32),
        ),
    )(x, wcat, atop, abot)


def _tc2(agg, b1cat, w2p, a2t, a2b):
    return pl.pallas_call(
        _tc2_body,
        out_shape=(
            jax.ShapeDtypeStruct((N, D2), _f32),
            jax.ShapeDtypeStruct((N, NHEAD), _f32),
            jax.ShapeDtypeStruct((N, NHEAD), _f32),
            jax.ShapeDtypeStruct((N, NHEAD), _f32),
        ),
    )(agg, b1cat, w2p, a2t, a2b)


def _tc3(agg2, b2p):
    return pl.pallas_call(
        _tc3_body,
        out_shape=jax.ShapeDtypeStruct((N, NCLASS), _f32),
    )(agg2, b2p)


def _make_sc(K, D):
    HB = D // K
    mesh = plsc.VectorSubcoreMesh(core_axis_name="c", subcore_axis_name="s")

    @functools.partial(
        pl.kernel,
        mesh=mesh,
        compiler_params=pltpu.CompilerParams(
            needs_layout_passes=False, use_tc_tiling_on_sc=False),
        out_type=jax.ShapeDtypeStruct((N * D,), _f32),
        scratch_types=[
            pltpu.VMEM((N * NHEAD,), _f32),
            pltpu.VMEM((RPT * NHEAD,), _f32),
            pltpu.VMEM((RPT * NHEAD,), _f32),
            pltpu.VMEM((RPT * NHEAD,), _f32),
            pltpu.VMEM((RPT * D,), _f32),
            pltpu.VMEM((C,), jnp.int32),
            pltpu.VMEM((C,), jnp.int32),
            pltpu.VMEM((C,), jnp.int32),
            pltpu.VMEM((C,), jnp.int32),
            pltpu.VMEM((C, D), _f32),
            pltpu.VMEM((C, D), _f32),
            pltpu.VMEM((64,), jnp.int32),
            pltpu.SemaphoreType.DMA,
            pltpu.SemaphoreType.DMA,
        ],
    )
    def sc_kernel(ekey_hbm, tgt_hbm, s_hbm, t_hbm, m_hbm, h_hbm, starts_hbm,
                  out_hbm, t_v, s_v, m_v, z_v, acc_v, kbuf0, kbuf1, ibuf0,
                  ibuf1, rowbuf0, rowbuf1, st_v, sem0, sem1):
        wid = lax.axis_index("s") * 2 + lax.axis_index("c")
        base = wid * RPT
        iota = lax.iota(jnp.int32, 16)

        pltpu.sync_copy(t_hbm, t_v)
        pltpu.sync_copy(s_hbm.at[pl.ds(base * NHEAD, RPT * NHEAD)], s_v)
        pltpu.sync_copy(m_hbm.at[pl.ds(base * NHEAD, RPT * NHEAD)], m_v)
        pltpu.sync_copy(starts_hbm, st_v)
        start = st_v[pl.ds(wid, 16)][0]
        end = st_v[pl.ds(wid + 1, 16)][0]

        zeros = jnp.zeros((16,), _f32)

        def _z1(i, _):
            z_v[pl.ds(i * 16, 16)] = zeros
            return 0
        lax.fori_loop(0, RPT * NHEAD // 16, _z1, 0)

        def _z2(i, _):
            acc_v[pl.ds(i * 16, 16)] = zeros
            return 0
        lax.fori_loop(0, RPT * D // 16, _z2, 0)

        j0 = start // C
        j1 = (end + C - 1) // C
        npairs = (j1 - j0 + 1) // 2

        def _decode(j, kb, v):
            kvec = kb[pl.ds(v * 16, 16)]
            eidx = (j * C + v * 16) + iota
            dup = (kvec >> 24) > 0
            valid = (eidx >= start) & (eidx < end) & jnp.logical_not(dup)
            srcl = jnp.clip((kvec >> 12) & (N - 1), base, base + RPT - 1) - base
            tgt = kvec & (N - 1)
            return valid, srcl, tgt

        def _edge_p(k, srcl, tgt, valid):
            kk = jnp.full((16,), k, jnp.int32)
            tval = plsc.load_gather(t_v, [tgt * NHEAD + kk])
            sval = plsc.load_gather(s_v, [srcl * NHEAD + kk])
            mval = plsc.load_gather(m_v, [srcl * NHEAD + kk])
            xx = sval + tval
            lg = jnp.where(xx > 0, xx, ALPHA * xx)
            p = jnp.exp(lg - mval)
            return kk, jnp.where(valid, p, 0.0)

        def _p1_load(jc, kb, sm):
            pltpu.async_copy(ekey_hbm.at[pl.ds(jc * C, C)], kb, sm)

        def _p1_wait(kb, sm):
            pltpu.make_async_copy(ekey_hbm.at[pl.ds(0, C)], kb, sm).wait()

        def _p1_proc(j, kb):
            def _vreg(v, _):
                valid, srcl, tgt = _decode(j, kb, v)
                ps = [_edge_p(k, srcl, tgt, valid) for k in range(K)]
                for kk, p in ps:
                    plsc.addupdate_scatter(
                        z_v, [srcl * NHEAD + kk], p, mask=valid)
                return 0
            lax.fori_loop(0, C // 16, _vreg, 0)

        _p1_load(j0, kbuf0, sem0)

        def _pair1(s_, _):
            j = j0 + 2 * s_
            _p1_load(jnp.minimum(j + 1, JLAST), kbuf1, sem1)
            _p1_wait(kbuf0, sem0)
            _p1_proc(j, kbuf0)
            _p1_load(jnp.minimum(j + 2, JLAST), kbuf0, sem0)
            _p1_wait(kbuf1, sem1)
            _p1_proc(j + 1, kbuf1)
            return 0

        lax.fori_loop(0, npairs, _pair1, 0)
        _p1_wait(kbuf0, sem0)

        def _p2_load(jc, kb, ib, rb, sm):
            pltpu.sync_copy(ekey_hbm.at[pl.ds(jc * C, C)], kb)
            pltpu.sync_copy(tgt_hbm.at[pl.ds(jc * C, C)], ib)
            pltpu.async_copy(h_hbm.at[ib], rb, sm)

        def _p2_wait(ib, rb, sm):
            pltpu.make_async_copy(h_hbm.at[ib], rb, sm).wait()

        def _p2_proc(j, kb, rb):
            def _vreg(v, _):
                valid, srcl, tgt = _decode(j, kb, v)
                ws = []
                for k in range(K):
                    kk, p = _edge_p(k, srcl, tgt, valid)
                    zval = plsc.load_gather(z_v, [srcl * NHEAD + kk])
                    ws.append(p / zval)
                off_vec = srcl * D
                for i in range(16):
                    off = off_vec[i]
                    row = v * 16 + i
                    vals = []
                    for k in range(K):
                        wsc = ws[k][i]
                        for c2 in range(HB // 16):
                            cc = k * HB + c2 * 16
                            vals.append((cc, rb[row, pl.ds(cc, 16)] * wsc))
                    for cc, val in vals:
                        plsc.addupdate(acc_v.at[pl.ds(off + cc, 16)], val)
                return 0
            lax.fori_loop(0, C // 16, _vreg, 0)

        _p2_load(j0, kbuf0, ibuf0, rowbuf0, sem0)

        def _pair2(s_, _):
            j = j0 + 2 * s_
            _p2_load(jnp.minimum(j + 1, JLAST), kbuf1, ibuf1, rowbuf1, sem1)
            _p2_wait(ibuf0, rowbuf0, sem0)
            _p2_proc(j, kbuf0, rowbuf0)
            _p2_load(jnp.minimum(j + 2, JLAST), kbuf0, ibuf0, rowbuf0, sem0)
            _p2_wait(ibuf1, rowbuf1, sem1)
            _p2_proc(j + 1, kbuf1, rowbuf1)
            return 0

        lax.fori_loop(0, npairs, _pair2, 0)
        _p2_wait(ibuf0, rowbuf0, sem0)

        pltpu.sync_copy(acc_v, out_hbm.at[pl.ds(base * D, RPT * D)])

    return sc_kernel


_sc_l1 = _make_sc(NHEAD, D1)
_sc_l2 = _make_sc(1, D2)


def kernel(x, edge_index, W1, a1, b1, W2, a2, b2):
    i32 = jnp.int32
    f32 = jnp.float32
    x = x.astype(f32)

    loop = jnp.arange(N, dtype=i32)
    srcs = jnp.concatenate([edge_index[0].astype(i32), loop])
    tgts = jnp.concatenate([edge_index[1].astype(i32), loop])
    key = jnp.sort(srcs * N + tgts)
    dupbit = jnp.concatenate(
        [jnp.zeros((1,), i32), (key[1:] == key[:-1]).astype(i32)])
    ekey = jnp.concatenate(
        [key | (dupbit << 24), jnp.full((EPAD - EP,), 1 << 24, i32)])
    tgtarr = jnp.concatenate(
        [key & (N - 1), jnp.zeros((EPAD - EP,), i32)])
    bounds = (jnp.arange(NT + 1, dtype=i32) * RPT) * N
    starts = jnp.searchsorted(key, bounds).astype(i32)
    starts = jnp.concatenate(
        [starts, jnp.full((64 - NT - 1,), EP, i32)])

    eye = jnp.eye(NHEAD, dtype=f32)
    w1cat = W1.astype(f32).transpose(1, 0, 2).reshape(NFEAT, D1)
    atop = (a1[:, :NHID, 0].astype(f32)[:, :, None]
            * eye[:, None, :]).reshape(D1, NHEAD)
    abot = (a1[:, NHID:, 0].astype(f32)[:, :, None]
            * eye[:, None, :]).reshape(D1, NHEAD)
    b1cat = b1.astype(f32).reshape(1, D1)
    w2p = jnp.concatenate(
        [W2.astype(f32), jnp.zeros((D1, D2 - NCLASS), f32)], axis=1)
    a2t = jnp.zeros((D2, NHEAD), f32).at[:NCLASS, :].set(
        a2[:NCLASS, 0].astype(f32)[:, None])
    a2b = jnp.zeros((D2, NHEAD), f32).at[:NCLASS, :].set(
        a2[NCLASS:, 0].astype(f32)[:, None])
    b2p = jnp.concatenate(
        [b2.astype(f32), jnp.zeros((D2 - NCLASS,), f32)]).reshape(1, D2)

    h1, s1, t1, m1 = _tc1(x, w1cat, atop, abot)
    agg1 = _sc_l1(ekey, tgtarr, s1.reshape(-1), t1.reshape(-1),
                  m1.reshape(-1), h1, starts).reshape(N, D1)
    h2, s2, t2, m2 = _tc2(agg1, b1cat, w2p, a2t, a2b)
    agg2 = _sc_l2(ekey, tgtarr, s2.reshape(-1), t2.reshape(-1),
                  m2.reshape(-1), h2, starts).reshape(N, D2)
    return _tc3(agg2, b2p)
